# baseline (speedup 1.0000x reference)
import functools

import jax
import jax.numpy as jnp
from jax import lax
from jax.experimental import pallas as pl
from jax.experimental.pallas import tpu as pltpu
from jax.experimental.pallas import tpu_sc as plsc

NC = 2
NS = 16
NW = NC * NS
LN = 16
SB = 4
PF = 3


def kernel(token_ids, weights):
    B, S = token_ids.shape
    V, D = weights.shape
    TBLK = B // 128
    assert TBLK == NW and D == 64 and S % SB == 0
    idx_t = jnp.swapaxes(token_ids, 0, 1).astype(jnp.int32)

    @functools.partial(
        pl.kernel,
        mesh=plsc.VectorSubcoreMesh(core_axis_name="c", subcore_axis_name="s"),
        out_type=jax.ShapeDtypeStruct((S * 8 * TBLK, 8, 128), jnp.float32),
        compiler_params=pltpu.CompilerParams(use_tc_tiling_on_sc=False,
                                             needs_layout_passes=False),
        scratch_types=[
            pltpu.VMEM((S, 128), jnp.int32),
            pltpu.VMEM((SB, 128, D), jnp.float32),
            pltpu.VMEM((SB, D, 128), jnp.float32),
            [pltpu.SemaphoreType.DMA] * SB,
            [pltpu.SemaphoreType.DMA] * SB,
        ],
    )
    def gather_k(idx_hbm, table_hbm, out_hbm, idx_v, rows_v, tile_v, gsem, psem):
        wid = lax.axis_index("s") * NC + lax.axis_index("c")
        pltpu.sync_copy(idx_hbm.at[:, pl.ds(wid * 128, 128)], idx_v)

        toks = [lax.iota(jnp.int32, LN) + (g * LN) for g in range(8)]

        def fire_gather(m, slot):
            pltpu.async_copy(table_hbm.at[idx_v.at[m]], rows_v.at[slot],
                             gsem[slot])

        def wait_gather(slot):
            pltpu.make_async_copy(table_hbm.at[idx_v.at[0]], rows_v.at[slot],
                                  gsem[slot]).wait()

        def fire_puts(s, slot):
            for tr in range(8):
                pltpu.async_copy(tile_v.at[slot, pl.ds(tr * 8, 8)],
                                 out_hbm.at[(s * 8 + tr) * TBLK + wid],
                                 psem[slot])

        def wait_puts(slot):
            for _ in range(8):
                pltpu.make_async_copy(tile_v.at[slot, pl.ds(0, 8)],
                                      out_hbm.at[0], psem[slot]).wait()

        def transpose(slot):
            for d in range(D):
                feat = jnp.full((LN,), d, jnp.int32)
                for g in range(8):
                    vals = plsc.load_gather(rows_v.at[slot], [toks[g], feat])
                    tile_v[slot, d, pl.ds(g * LN, LN)] = vals

        def step(j, b):
            m = j + PF

            @pl.when(m < S)
            def _():
                fire_gather(m, (b + PF) % SB)

            wait_gather(b)

            @pl.when(j >= SB)
            def _():
                wait_puts(b)

            transpose(b)
            fire_puts(j, b)

        for m in range(PF):
            fire_gather(m, m)

        def outer(g, carry):
            for b in range(SB):
                step(g * SB + b, b)
            return carry

        lax.fori_loop(0, S // SB, outer, 0)

        for b in range(SB):
            wait_puts(b)

    out = gather_k(idx_t, weights)
    o5 = out.reshape(S, 8, TBLK, 8, 128)
    return o5.transpose(2, 4, 0, 1, 3).reshape(B, S, D)


# speedup vs baseline: 1.6765x; 1.6765x over previous
import functools

import jax
import jax.numpy as jnp
from jax import lax
from jax.experimental import pallas as pl
from jax.experimental.pallas import tpu as pltpu
from jax.experimental.pallas import tpu_sc as plsc

NC = 2
NS = 16
NW = NC * NS
LN = 16
SB = 4
PF = 3


def kernel(token_ids, weights):
    B, S = token_ids.shape
    V, D = weights.shape
    TBLK = B // 128
    assert TBLK == NW and D == 64 and S % SB == 0
    idx_t = jnp.swapaxes(token_ids, 0, 1).astype(jnp.int32)

    @functools.partial(
        pl.kernel,
        mesh=plsc.VectorSubcoreMesh(core_axis_name="c", subcore_axis_name="s"),
        out_type=jax.ShapeDtypeStruct((S * 8 * TBLK, 8, 128), jnp.float32),
        compiler_params=pltpu.CompilerParams(use_tc_tiling_on_sc=False,
                                             needs_layout_passes=False),
        scratch_types=[
            pltpu.VMEM((S, 128), jnp.int32),
            pltpu.VMEM((SB, 128, D), jnp.float32),
            pltpu.VMEM((SB, D, 128), jnp.float32),
            [pltpu.SemaphoreType.DMA] * SB,
            [pltpu.SemaphoreType.DMA] * SB,
        ],
    )
    def gather_k(idx_hbm, table_hbm, out_hbm, idx_v, rows_v, tile_v, gsem, psem):
        wid = lax.axis_index("s") * NC + lax.axis_index("c")
        pltpu.sync_copy(idx_hbm.at[:, pl.ds(wid * 128, 128)], idx_v)

        iota = lax.iota(jnp.int32, LN)
        rvecs = [(iota + r) % LN for r in range(LN)]

        def fire_gather(m, slot):
            pltpu.async_copy(table_hbm.at[idx_v.at[m]], rows_v.at[slot],
                             gsem[slot])

        def wait_gather(slot):
            pltpu.make_async_copy(table_hbm.at[idx_v.at[0]], rows_v.at[slot],
                                  gsem[slot]).wait()

        def fire_puts(s, slot):
            for tr in range(8):
                pltpu.async_copy(tile_v.at[slot, pl.ds(tr * 8, 8)],
                                 out_hbm.at[(s * 8 + tr) * TBLK + wid],
                                 psem[slot])

        def wait_puts(slot):
            for _ in range(8):
                pltpu.make_async_copy(tile_v.at[slot, pl.ds(0, 8)],
                                      out_hbm.at[0], psem[slot]).wait()

        def transpose(slot):
            rows_2d = rows_v.at[slot]
            tile_2d = tile_v.at[slot]

            def bt_body(bt, carry):
                tok = iota + bt * LN
                for k in range(D // LN):
                    for r in range(LN):
                        feat = rvecs[r] + (k * LN)
                        vals = plsc.load_gather(rows_2d, [tok, feat])
                        plsc.store_scatter(tile_2d, [feat, tok], vals)
                return carry

            lax.fori_loop(0, 128 // LN, bt_body, 0)

        def step(j, b):
            m = j + PF

            @pl.when(m < S)
            def _():
                fire_gather(m, (b + PF) % SB)

            wait_gather(b)

            @pl.when(j >= SB)
            def _():
                wait_puts(b)

            transpose(b)
            fire_puts(j, b)

        for m in range(PF):
            fire_gather(m, m)

        def outer(g, carry):
            for b in range(SB):
                step(g * SB + b, b)
            return carry

        lax.fori_loop(0, S // SB, outer, 0)

        for b in range(SB):
            wait_puts(b)

    out = gather_k(idx_t, weights)
    o5 = out.reshape(S, 8, TBLK, 8, 128)
    return o5.transpose(2, 4, 0, 1, 3).reshape(B, S, D)


# speedup vs baseline: 2.3690x; 1.4131x over previous
import functools

import jax
import jax.numpy as jnp
from jax import lax
from jax.experimental import pallas as pl
from jax.experimental.pallas import tpu as pltpu
from jax.experimental.pallas import tpu_sc as plsc

NC = 2
NS = 16
NW = NC * NS
LN = 16
SB = 4
PF = 3


def kernel(token_ids, weights):
    B, S = token_ids.shape
    V, D = weights.shape
    TBLK = B // 128
    assert TBLK == NW and D == 64 and S % SB == 0
    idx_t = jnp.swapaxes(token_ids, 0, 1).astype(jnp.int32)

    @functools.partial(
        pl.kernel,
        mesh=plsc.VectorSubcoreMesh(core_axis_name="c", subcore_axis_name="s"),
        out_type=jax.ShapeDtypeStruct((S * 8 * TBLK, 8, 128), jnp.float32),
        compiler_params=pltpu.CompilerParams(use_tc_tiling_on_sc=False,
                                             needs_layout_passes=False),
        scratch_types=[
            pltpu.VMEM((S, 128), jnp.int32),
            pltpu.VMEM((SB, 128, D), jnp.float32),
            pltpu.VMEM((SB, D, 128), jnp.float32),
            [pltpu.SemaphoreType.DMA] * SB,
            [pltpu.SemaphoreType.DMA] * SB,
        ],
    )
    def gather_k(idx_hbm, table_hbm, out_hbm, idx_v, rows_v, tile_v, gsem, psem):
        wid = lax.axis_index("s") * NC + lax.axis_index("c")
        pltpu.sync_copy(idx_hbm.at[:, pl.ds(wid * 128, 128)], idx_v)

        iota = lax.iota(jnp.int32, LN)
        rvecs = [(iota + r) % LN for r in range(LN)]

        def fire_gather(m, slot):
            pltpu.async_copy(table_hbm.at[idx_v.at[m]], rows_v.at[slot],
                             gsem[slot])

        def wait_gather(slot):
            pltpu.make_async_copy(table_hbm.at[idx_v.at[0]], rows_v.at[slot],
                                  gsem[slot]).wait()

        def fire_puts(s, slot):
            for tr in range(8):
                pltpu.async_copy(tile_v.at[slot, pl.ds(tr * 8, 8)],
                                 out_hbm.at[(s * 8 + tr) * TBLK + wid],
                                 psem[slot])

        def wait_puts(slot):
            for _ in range(8):
                pltpu.make_async_copy(tile_v.at[slot, pl.ds(0, 8)],
                                      out_hbm.at[0], psem[slot]).wait()

        def transpose(slot):
            rows_2d = rows_v.at[slot]
            tile_2d = tile_v.at[slot]

            def bt_body(bt, carry):
                tok = iota + bt * LN
                for k in range(D // LN):
                    feats = [rvecs[r] + (k * LN) for r in range(LN)]
                    vals = [plsc.load_gather(rows_2d, [tok, feats[r]])
                            for r in range(LN)]
                    for r in range(LN):
                        plsc.store_scatter(tile_2d, [feats[r], tok], vals[r])
                return carry

            lax.fori_loop(0, 128 // LN, bt_body, 0)

        def step(j, b):
            m = j + PF

            @pl.when(m < S)
            def _():
                fire_gather(m, (b + PF) % SB)

            wait_gather(b)

            @pl.when(j >= SB)
            def _():
                wait_puts(b)

            transpose(b)
            fire_puts(j, b)

        for m in range(PF):
            fire_gather(m, m)

        def outer(g, carry):
            for b in range(SB):
                step(g * SB + b, b)
            return carry

        lax.fori_loop(0, S // SB, outer, 0)

        for b in range(SB):
            wait_puts(b)

    out = gather_k(idx_t, weights)
    o5 = out.reshape(S, 8, TBLK, 8, 128)
    return o5.transpose(2, 4, 0, 1, 3).reshape(B, S, D)
